# baseline (speedup 1.0000x reference)
import functools

import jax
import jax.numpy as jnp
from jax import lax
from jax.experimental import pallas as pl
from jax.experimental.pallas import tpu as pltpu
from jax.experimental.pallas import tpu_sc as plsc

_VOCAB = 1000000
_D = 64
_HID = 64
_NIDX = 16384

_NC = 2
_NS = 16
_NW = _NC * _NS
_PER_W = _NIDX // _NW
_LANES = 16

_TILE_V = 32768
_NT = (_VOCAB + _TILE_V - 1) // _TILE_V

_TILE_F = 65536
_NF = (_VOCAB + _TILE_F - 1) // _TILE_F


def _sc_gather_sum(idx_hbm, emb_hbm, out_hbm, idx_v, pair_v, rows_v, acc_v,
                   sem):
    wid = lax.axis_index("s") * _NC + lax.axis_index("c")
    base = wid * _PER_W
    pltpu.sync_copy(idx_hbm.at[pl.ds(base, _PER_W)], idx_v)
    for c in range(_PER_W // _LANES):
        pair_v[pl.ds(c * _LANES, _LANES)] = (
            lax.shift_right_logical(idx_v[pl.ds(c * _LANES, _LANES)], 1))
    pltpu.async_copy(emb_hbm.at[pair_v], rows_v, sem).wait()

    zeros = jnp.zeros((_LANES,), jnp.float32)

    def body(g, carry):
        offs = (idx_v[pl.ds(g * _LANES, _LANES)] & 1) * _D
        for l in range(_LANES):
            off = offs[l]
            carry = tuple(
                carry[k] + rows_v[g * _LANES + l,
                                  pl.ds(off + k * _LANES, _LANES)]
                for k in range(_D // _LANES)
            )
        return carry

    acc = lax.fori_loop(0, _PER_W // _LANES, body,
                        (zeros,) * (_D // _LANES))
    for k in range(_D // _LANES):
        acc_v[0, pl.ds(k * _LANES, _LANES)] = acc[k]
    pltpu.sync_copy(acc_v, out_hbm.at[pl.ds(wid, 1)])


def _gather_sum(idx, emb_pairs):
    fn = functools.partial(
        pl.kernel,
        mesh=plsc.VectorSubcoreMesh(core_axis_name="c", subcore_axis_name="s"),
        out_type=jax.ShapeDtypeStruct((_NW, _D), jnp.float32),
        scratch_types=[
            pltpu.VMEM((_PER_W,), jnp.int32),
            pltpu.VMEM((_PER_W,), jnp.int32),
            pltpu.VMEM((_PER_W, 2 * _D), jnp.float32),
            pltpu.VMEM((1, _D), jnp.float32),
            pltpu.SemaphoreType.DMA,
        ],
    )(_sc_gather_sum)
    return fn(idx, emb_pairs)


def _tc_body(parts_ref, w1_ref, b1_ref, w2_ref, b2_ref, out_ref, lse_ref,
             stats_ref):
    i = pl.program_id(0)

    @pl.when(i == 0)
    def _():
        stats_ref[0] = -jnp.inf
        stats_ref[1] = 0.0

    v = jnp.sum(parts_ref[...], axis=0, keepdims=True)
    h = lax.dot_general(v, w1_ref[...], (((1,), (1,)), ((), ())),
                        preferred_element_type=jnp.float32)
    h = jnp.maximum(h + b1_ref[...], 0.0)
    logits = lax.dot_general(h, w2_ref[...], (((1,), (0,)), ((), ())),
                             preferred_element_type=jnp.float32)
    logits = logits + b2_ref[...]
    out_ref[...] = logits

    valid = _VOCAB - i * _TILE_V
    lane = lax.broadcasted_iota(jnp.int32, (1, _TILE_V), 1)
    logits_m = jnp.where(lane < valid, logits, -jnp.inf)

    m_old = stats_ref[0]
    m_new = jnp.maximum(m_old, jnp.max(logits_m))
    stats_ref[1] = (stats_ref[1] * jnp.exp(m_old - m_new)
                    + jnp.sum(jnp.exp(logits_m - m_new)))
    stats_ref[0] = m_new

    @pl.when(i == _NT - 1)
    def _():
        lse_ref[...] = jnp.full((1, 128), stats_ref[0] + jnp.log(stats_ref[1]),
                                jnp.float32)


def _tc_logits_lse(parts, w1, b1, w2t, b2):
    return pl.pallas_call(
        _tc_body,
        grid=(_NT,),
        in_specs=[
            pl.BlockSpec((_NW, _D), lambda i: (0, 0)),
            pl.BlockSpec((_HID, _D), lambda i: (0, 0)),
            pl.BlockSpec((1, _HID), lambda i: (0, 0)),
            pl.BlockSpec((_HID, _TILE_V), lambda i: (0, i)),
            pl.BlockSpec((1, _TILE_V), lambda i: (0, i)),
        ],
        out_specs=[
            pl.BlockSpec((1, _TILE_V), lambda i: (0, i)),
            pl.BlockSpec((1, 128), lambda i: (0, 0)),
        ],
        out_shape=[
            jax.ShapeDtypeStruct((1, _VOCAB), jnp.float32),
            jax.ShapeDtypeStruct((1, 128), jnp.float32),
        ],
        scratch_shapes=[pltpu.SMEM((2,), jnp.float32)],
        compiler_params=pltpu.CompilerParams(
            dimension_semantics=("arbitrary",)),
    )(parts, w1, b1, w2t, b2)


def _sub_body(logits_ref, lse_ref, out_ref):
    out_ref[...] = logits_ref[...] - lse_ref[0, 0]


def _tc_subtract(logits, lse):
    return pl.pallas_call(
        _sub_body,
        grid=(_NF,),
        in_specs=[
            pl.BlockSpec((1, _TILE_F), lambda i: (0, i)),
            pl.BlockSpec((1, 128), lambda i: (0, 0)),
        ],
        out_specs=pl.BlockSpec((1, _TILE_F), lambda i: (0, i)),
        out_shape=jax.ShapeDtypeStruct((1, _VOCAB), jnp.float32),
        compiler_params=pltpu.CompilerParams(
            dimension_semantics=("arbitrary",)),
    )(logits, lse)


def kernel(inputs, embeddings, W1, b1, W2, b2):
    parts = _gather_sum(inputs, embeddings.reshape(_VOCAB // 2, 2 * _D))
    logits, lse = _tc_logits_lse(parts, W1, b1.reshape(1, _HID),
                                 jnp.swapaxes(W2, 0, 1),
                                 b2.reshape(1, _VOCAB))
    return _tc_subtract(logits, lse)


# speedup vs baseline: 1.5684x; 1.5684x over previous
import functools

import jax
import jax.numpy as jnp
from jax import lax
from jax.experimental import pallas as pl
from jax.experimental.pallas import tpu as pltpu
from jax.experimental.pallas import tpu_sc as plsc

_VOCAB = 1000000
_D = 64
_HID = 64
_NIDX = 16384

_NC = 2
_NS = 16
_NW = _NC * _NS
_PER_W = _NIDX // _NW
_LANES = 16

_TILE_V = 32768
_NT = (_VOCAB + _TILE_V - 1) // _TILE_V

_TILE_F = 65536
_NF = (_VOCAB + _TILE_F - 1) // _TILE_F


def _sc_gather_sum(idx_hbm, emb_hbm, out_hbm, idx_v, rows_v, acc_v, sem):
    wid = lax.axis_index("s") * _NC + lax.axis_index("c")
    base = wid * _PER_W
    pltpu.sync_copy(idx_hbm.at[pl.ds(base, _PER_W)], idx_v)

    def fire(g, carry):
        chunk = idx_v[pl.ds(g * _LANES, _LANES)]
        for l in range(_LANES):
            pltpu.async_copy(
                emb_hbm.at[pl.ds(chunk[l], 1)],
                rows_v.at[pl.ds(g * _LANES + l, 1)],
                sem,
            )
        return carry

    lax.fori_loop(0, _PER_W // _LANES, fire, 0)
    pltpu.make_async_copy(emb_hbm.at[pl.ds(0, _PER_W)], rows_v, sem).wait()

    zeros = jnp.zeros((_LANES,), jnp.float32)

    def body(j, carry):
        return tuple(
            carry[k] + rows_v[j, pl.ds(k * _LANES, _LANES)]
            for k in range(_D // _LANES)
        )

    acc = lax.fori_loop(0, _PER_W, body, (zeros,) * (_D // _LANES))
    for k in range(_D // _LANES):
        acc_v[0, pl.ds(k * _LANES, _LANES)] = acc[k]
    pltpu.sync_copy(acc_v, out_hbm.at[pl.ds(wid, 1)])


def _gather_sum(idx, emb):
    fn = functools.partial(
        pl.kernel,
        mesh=plsc.VectorSubcoreMesh(core_axis_name="c", subcore_axis_name="s"),
        out_type=jax.ShapeDtypeStruct((_NW, _D), jnp.float32),
        scratch_types=[
            pltpu.VMEM((_PER_W,), jnp.int32),
            pltpu.VMEM((_PER_W, _D), jnp.float32),
            pltpu.VMEM((1, _D), jnp.float32),
            pltpu.SemaphoreType.DMA,
        ],
    )(_sc_gather_sum)
    return fn(idx, emb)


def _tc_body(parts_ref, w1_ref, b1_ref, w2_ref, b2_ref, out_ref, lse_ref,
             stats_ref):
    i = pl.program_id(0)

    @pl.when(i == 0)
    def _():
        stats_ref[0] = -jnp.inf
        stats_ref[1] = 0.0

    v = jnp.sum(parts_ref[...], axis=0, keepdims=True)
    h = lax.dot_general(v, w1_ref[...], (((1,), (1,)), ((), ())),
                        preferred_element_type=jnp.float32)
    h = jnp.maximum(h + b1_ref[...], 0.0)
    logits = lax.dot_general(h, w2_ref[...], (((1,), (0,)), ((), ())),
                             preferred_element_type=jnp.float32)
    logits = logits + b2_ref[...]
    out_ref[...] = logits

    valid = _VOCAB - i * _TILE_V
    lane = lax.broadcasted_iota(jnp.int32, (1, _TILE_V), 1)
    logits_m = jnp.where(lane < valid, logits, -jnp.inf)

    m_old = stats_ref[0]
    m_new = jnp.maximum(m_old, jnp.max(logits_m))
    stats_ref[1] = (stats_ref[1] * jnp.exp(m_old - m_new)
                    + jnp.sum(jnp.exp(logits_m - m_new)))
    stats_ref[0] = m_new

    @pl.when(i == _NT - 1)
    def _():
        lse_ref[...] = jnp.full((1, 128), stats_ref[0] + jnp.log(stats_ref[1]),
                                jnp.float32)


def _tc_logits_lse(parts, w1, b1, w2t, b2):
    return pl.pallas_call(
        _tc_body,
        grid=(_NT,),
        in_specs=[
            pl.BlockSpec((_NW, _D), lambda i: (0, 0)),
            pl.BlockSpec((_HID, _D), lambda i: (0, 0)),
            pl.BlockSpec((1, _HID), lambda i: (0, 0)),
            pl.BlockSpec((_HID, _TILE_V), lambda i: (0, i)),
            pl.BlockSpec((1, _TILE_V), lambda i: (0, i)),
        ],
        out_specs=[
            pl.BlockSpec((1, _TILE_V), lambda i: (0, i)),
            pl.BlockSpec((1, 128), lambda i: (0, 0)),
        ],
        out_shape=[
            jax.ShapeDtypeStruct((1, _VOCAB), jnp.float32),
            jax.ShapeDtypeStruct((1, 128), jnp.float32),
        ],
        scratch_shapes=[pltpu.SMEM((2,), jnp.float32)],
        compiler_params=pltpu.CompilerParams(
            dimension_semantics=("arbitrary",)),
    )(parts, w1, b1, w2t, b2)


def _sub_body(logits_ref, lse_ref, out_ref):
    out_ref[...] = logits_ref[...] - lse_ref[0, 0]


def _tc_subtract(logits, lse):
    return pl.pallas_call(
        _sub_body,
        grid=(_NF,),
        in_specs=[
            pl.BlockSpec((1, _TILE_F), lambda i: (0, i)),
            pl.BlockSpec((1, 128), lambda i: (0, 0)),
        ],
        out_specs=pl.BlockSpec((1, _TILE_F), lambda i: (0, i)),
        out_shape=jax.ShapeDtypeStruct((1, _VOCAB), jnp.float32),
        compiler_params=pltpu.CompilerParams(
            dimension_semantics=("arbitrary",)),
    )(logits, lse)


def kernel(inputs, embeddings, W1, b1, W2, b2):
    parts = _gather_sum(inputs, embeddings)
    logits, lse = _tc_logits_lse(parts, W1, b1.reshape(1, _HID),
                                 jnp.swapaxes(W2, 0, 1),
                                 b2.reshape(1, _VOCAB))
    return _tc_subtract(logits, lse)
